# baseline (speedup 1.0000x reference)
import jax
import jax.numpy as jnp
from jax.experimental import pallas as pl
from jax.experimental.pallas import tpu as pltpu

_KS = 7
_H = 384
_W = 384
_B = 8
_C = 3
_TOPN = int(_H * _W * 0.009)
_ONE_BITS = 0x3F800000


def _reflect_pad_rows(x):
    return jnp.concatenate(
        [x[3:4], x[2:3], x[1:2], x,
         x[_H - 2:_H - 1], x[_H - 3:_H - 2], x[_H - 4:_H - 3]], axis=0)


def _reflect_pad_cols(x):
    return jnp.concatenate(
        [x[:, 3:4], x[:, 2:3], x[:, 1:2], x,
         x[:, _W - 2:_W - 1], x[:, _W - 3:_W - 2], x[:, _W - 4:_W - 3]],
        axis=1)


def _window_min(dcc):
    p = _reflect_pad_rows(dcc)
    w2 = jnp.minimum(p[0:_H + 5], p[1:_H + 6])
    w4 = jnp.minimum(w2[0:_H + 3], w2[2:_H + 5])
    m = jnp.minimum(w4[0:_H], w4[3:_H + 3])
    q = _reflect_pad_cols(m)
    v2 = jnp.minimum(q[:, 0:_W + 5], q[:, 1:_W + 6])
    v4 = jnp.minimum(v2[:, 0:_W + 3], v2[:, 2:_W + 5])
    return jnp.minimum(v4[:, 0:_W], v4[:, 3:_W + 3])


def _dcp_kernel(img_hbm, out_ref, img_ref, dc_ref, sem0, sem1):
    sems = (sem0, sem1)

    def copy(b):
        return pltpu.make_async_copy(
            img_hbm.at[b], img_ref.at[b], sems[b % 2])

    copy(0).start()
    copy(1).start()
    for b in range(_B):
        copy(b).wait()
        if b + 2 < _B:
            copy(b + 2).start()
        img = img_ref[b]
        dcc = jnp.minimum(jnp.minimum(img[0], img[1]), img[2])
        dc_ref[b] = jax.lax.bitcast_convert_type(_window_min(dcc), jnp.int32)

    def bs_val(_, state):
        lo, hi = state
        new_lo = []
        new_hi = []
        for b in range(_B):
            mid = (lo[b] + hi[b]) // 2
            cnt_lt = jnp.sum(
                jax.lax.shift_right_logical(dc_ref[b] - mid, 31))
            ok = (_H * _W - cnt_lt) >= _TOPN
            new_lo.append(jnp.where(ok, mid, lo[b]))
            new_hi.append(jnp.where(ok, hi[b], mid))
        return (tuple(new_lo), tuple(new_hi))

    zeros = tuple(jnp.int32(0) for _ in range(_B))
    ones = tuple(jnp.int32(_ONE_BITS) for _ in range(_B))
    t, _ = jax.lax.fori_loop(0, 30, bs_val, (zeros, ones))

    m = []
    rowcnt = []
    for b in range(_B):
        bits = dc_ref[b]
        count_gt = jnp.sum((bits > t[b]).astype(jnp.int32))
        m.append(_TOPN - count_gt)
        rowcnt.append(jnp.sum((bits == t[b]).astype(jnp.int32), axis=1,
                              keepdims=True))

    riota = jax.lax.broadcasted_iota(jnp.int32, (_H, 1), 0)

    def bs_row(_, state):
        lo, hi = state
        new_lo = []
        new_hi = []
        for b in range(_B):
            mid = (lo[b] + hi[b]) // 2
            cnt = jnp.sum(jnp.where(riota <= mid, rowcnt[b], 0))
            ok = cnt >= m[b]
            new_lo.append(jnp.where(ok, lo[b], mid))
            new_hi.append(jnp.where(ok, mid, hi[b]))
        return (tuple(new_lo), tuple(new_hi))

    negs = tuple(jnp.int32(-1) for _ in range(_B))
    tops = tuple(jnp.int32(_H - 1) for _ in range(_B))
    _, rstar = jax.lax.fori_loop(0, 9, bs_row, (negs, tops))

    ciota = jax.lax.broadcasted_iota(jnp.int32, (1, _W), 1)
    eq_row = []
    mrow = []
    for b in range(_B):
        cnt_lt = jnp.sum(jnp.where(riota < rstar[b], rowcnt[b], 0))
        mrow.append(m[b] - cnt_lt)
        row = dc_ref[b, pl.ds(rstar[b], 1), :]
        eq_row.append(row == t[b])

    def bs_col(_, state):
        lo, hi = state
        new_lo = []
        new_hi = []
        for b in range(_B):
            mid = (lo[b] + hi[b]) // 2
            cnt = jnp.sum((eq_row[b] & (ciota <= mid)).astype(jnp.int32))
            ok = cnt >= mrow[b]
            new_lo.append(jnp.where(ok, lo[b], mid))
            new_hi.append(jnp.where(ok, mid, hi[b]))
        return (tuple(new_lo), tuple(new_hi))

    ctops = tuple(jnp.int32(_W - 1) for _ in range(_B))
    _, cstar = jax.lax.fori_loop(0, 9, bs_col, (negs, ctops))

    cut = [rstar[b] * _W + cstar[b] for b in range(_B)]

    idx = (jax.lax.broadcasted_iota(jnp.int32, (_H, _W), 0) * _W
           + jax.lax.broadcasted_iota(jnp.int32, (_H, _W), 1))

    total = 0.0
    for b in range(_B):
        bits = dc_ref[b]
        mask = (bits > t[b]) | ((bits == t[b]) & (idx <= cut[b]))
        for c in range(_C):
            mx = jnp.max(jnp.where(mask, img_ref[b, c], -1.0))
            total = total + jnp.minimum(mx, 0.89)
    out_ref[:, :] = jnp.full((1, 1), total / (_B * _C), jnp.float32)


def kernel(image):
    out = pl.pallas_call(
        _dcp_kernel,
        grid=(1,),
        in_specs=[pl.BlockSpec(memory_space=pl.ANY)],
        out_specs=pl.BlockSpec((1, 1), lambda i: (0, 0)),
        out_shape=jax.ShapeDtypeStruct((1, 1), jnp.float32),
        scratch_shapes=[
            pltpu.VMEM((_B, _C, _H, _W), jnp.float32),
            pltpu.VMEM((_B, _H, _W), jnp.int32),
            pltpu.SemaphoreType.DMA,
            pltpu.SemaphoreType.DMA,
        ],
    )(image)
    return out[0, 0]


# speedup vs baseline: 1.0354x; 1.0354x over previous
import jax
import jax.numpy as jnp
from jax.experimental import pallas as pl
from jax.experimental.pallas import tpu as pltpu

_KS = 7
_H = 384
_W = 384
_B = 8
_C = 3
_TOPN = int(_H * _W * 0.009)
_ONE_BITS = 0x3F800000


def _reflect_pad_rows(x):
    return jnp.concatenate(
        [x[3:4], x[2:3], x[1:2], x,
         x[_H - 2:_H - 1], x[_H - 3:_H - 2], x[_H - 4:_H - 3]], axis=0)


def _reflect_pad_cols(x):
    return jnp.concatenate(
        [x[:, 3:4], x[:, 2:3], x[:, 1:2], x,
         x[:, _W - 2:_W - 1], x[:, _W - 3:_W - 2], x[:, _W - 4:_W - 3]],
        axis=1)


def _window_min(dcc):
    p = _reflect_pad_rows(dcc)
    w2 = jnp.minimum(p[0:_H + 5], p[1:_H + 6])
    w4 = jnp.minimum(w2[0:_H + 3], w2[2:_H + 5])
    m = jnp.minimum(w4[0:_H], w4[3:_H + 3])
    q = _reflect_pad_cols(m)
    v2 = jnp.minimum(q[:, 0:_W + 5], q[:, 1:_W + 6])
    v4 = jnp.minimum(v2[:, 0:_W + 3], v2[:, 2:_W + 5])
    return jnp.minimum(v4[:, 0:_W], v4[:, 3:_W + 3])


def _dcp_kernel(img_hbm, out_ref, img_ref, dc_ref, sem0, sem1):
    sems = (sem0, sem1)

    def copy(b):
        return pltpu.make_async_copy(
            img_hbm.at[b], img_ref.at[b], sems[b % 2])

    copy(0).start()
    copy(1).start()
    for b in range(_B):
        copy(b).wait()
        if b + 2 < _B:
            copy(b + 2).start()
        img = img_ref[b]
        dcc = jnp.minimum(jnp.minimum(img[0], img[1]), img[2])
        dc_ref[b] = jax.lax.bitcast_convert_type(_window_min(dcc), jnp.int32)

    def bs_val_step(state):
        lo, hi = state
        new_lo = []
        new_hi = []
        for b in range(_B):
            mid = (lo[b] + hi[b]) // 2
            cnt = jnp.sum((dc_ref[b] >= mid).astype(jnp.int32))
            ok = cnt >= _TOPN
            new_lo.append(jnp.where(ok, mid, lo[b]))
            new_hi.append(jnp.where(ok, hi[b], mid))
        return (tuple(new_lo), tuple(new_hi))

    def bs_val(_, state):
        return bs_val_step(bs_val_step(state))

    zeros = tuple(jnp.int32(0) for _ in range(_B))
    ones = tuple(jnp.int32(_ONE_BITS) for _ in range(_B))
    t, _ = jax.lax.fori_loop(0, 15, bs_val, (zeros, ones))

    m = []
    rowcnt = []
    for b in range(_B):
        bits = dc_ref[b]
        count_gt = jnp.sum((bits > t[b]).astype(jnp.int32))
        m.append(_TOPN - count_gt)
        rowcnt.append(jnp.sum((bits == t[b]).astype(jnp.int32), axis=1,
                              keepdims=True))

    riota = jax.lax.broadcasted_iota(jnp.int32, (_H, 1), 0)

    def bs_row(_, state):
        lo, hi = state
        new_lo = []
        new_hi = []
        for b in range(_B):
            mid = (lo[b] + hi[b]) // 2
            cnt = jnp.sum(jnp.where(riota <= mid, rowcnt[b], 0))
            ok = cnt >= m[b]
            new_lo.append(jnp.where(ok, lo[b], mid))
            new_hi.append(jnp.where(ok, mid, hi[b]))
        return (tuple(new_lo), tuple(new_hi))

    negs = tuple(jnp.int32(-1) for _ in range(_B))
    tops = tuple(jnp.int32(_H - 1) for _ in range(_B))
    _, rstar = jax.lax.fori_loop(0, 9, bs_row, (negs, tops))

    ciota = jax.lax.broadcasted_iota(jnp.int32, (1, _W), 1)
    eq_row = []
    mrow = []
    for b in range(_B):
        cnt_lt = jnp.sum(jnp.where(riota < rstar[b], rowcnt[b], 0))
        mrow.append(m[b] - cnt_lt)
        row = dc_ref[b, pl.ds(rstar[b], 1), :]
        eq_row.append(row == t[b])

    def bs_col(_, state):
        lo, hi = state
        new_lo = []
        new_hi = []
        for b in range(_B):
            mid = (lo[b] + hi[b]) // 2
            cnt = jnp.sum((eq_row[b] & (ciota <= mid)).astype(jnp.int32))
            ok = cnt >= mrow[b]
            new_lo.append(jnp.where(ok, lo[b], mid))
            new_hi.append(jnp.where(ok, mid, hi[b]))
        return (tuple(new_lo), tuple(new_hi))

    ctops = tuple(jnp.int32(_W - 1) for _ in range(_B))
    _, cstar = jax.lax.fori_loop(0, 9, bs_col, (negs, ctops))

    cut = [rstar[b] * _W + cstar[b] for b in range(_B)]

    idx = (jax.lax.broadcasted_iota(jnp.int32, (_H, _W), 0) * _W
           + jax.lax.broadcasted_iota(jnp.int32, (_H, _W), 1))

    total = 0.0
    for b in range(_B):
        bits = dc_ref[b]
        mask = (bits > t[b]) | ((bits == t[b]) & (idx <= cut[b]))
        for c in range(_C):
            mx = jnp.max(jnp.where(mask, img_ref[b, c], -1.0))
            total = total + jnp.minimum(mx, 0.89)
    out_ref[:, :] = jnp.full((1, 1), total / (_B * _C), jnp.float32)


def kernel(image):
    out = pl.pallas_call(
        _dcp_kernel,
        grid=(1,),
        in_specs=[pl.BlockSpec(memory_space=pl.ANY)],
        out_specs=pl.BlockSpec((1, 1), lambda i: (0, 0)),
        out_shape=jax.ShapeDtypeStruct((1, 1), jnp.float32),
        scratch_shapes=[
            pltpu.VMEM((_B, _C, _H, _W), jnp.float32),
            pltpu.VMEM((_B, _H, _W), jnp.int32),
            pltpu.SemaphoreType.DMA,
            pltpu.SemaphoreType.DMA,
        ],
    )(image)
    return out[0, 0]
